# baseline (speedup 1.0000x reference)
import functools

import jax
import jax.numpy as jnp
from jax import lax
from jax.experimental import pallas as pl
from jax.experimental.pallas import tpu as pltpu

ALPHA = 3.0
TOPK = 32

_EXP_ONE = 0x3F800000
_MANT_MASK = 0x007FFFFF
_HI_INIT = 0x3F814800


def _rotl(x, r):
    return (x << r) | ((x >> (32 - r)) & ((1 << r) - 1))


def _threefry2x32(x0, x1):
    k0 = jnp.int32(0)
    k1 = jnp.int32(42)
    k2 = k0 ^ k1 ^ jnp.int32(0x1BD11BDA)
    ks = (k0, k1, k2)
    rotations = ((13, 15, 26, 6), (17, 29, 16, 24))
    x0 = x0 + ks[0]
    x1 = x1 + ks[1]
    for i in range(5):
        for r in rotations[i % 2]:
            x0 = x0 + x1
            x1 = _rotl(x1, r)
            x1 = x0 ^ x1
        x0 = x0 + ks[(i + 1) % 3]
        x1 = x1 + ks[(i + 2) % 3] + jnp.int32(i + 1)
    return x0, x1


def _noise_tile(r0, br, n):
    row = r0 + lax.broadcasted_iota(jnp.int32, (br, n), 0)
    col = lax.broadcasted_iota(jnp.int32, (br, n), 1)
    f = row * n + col
    o0, o1 = _threefry2x32(jnp.zeros_like(f), f)
    bits = o0 ^ o1
    fb = ((bits >> 9) & _MANT_MASK) | jnp.int32(_EXP_ONE)
    u = lax.bitcast_convert_type(fb, jnp.float32) - jnp.float32(1.0)
    return u * jnp.float32(0.01)


def _prologue_body(x1_ref, x2_ref, w1_ref, b1_ref, w2_ref, b2_ref, n1_ref, n2_ref):
    dn = (((1,), (1,)), ((), ()))
    h1 = lax.dot_general(x1_ref[...], w1_ref[...], dn,
                         preferred_element_type=jnp.float32) + b1_ref[...]
    n1_ref[...] = jnp.tanh(jnp.float32(ALPHA) * h1)
    h2 = lax.dot_general(x2_ref[...], w2_ref[...], dn,
                         preferred_element_type=jnp.float32) + b2_ref[...]
    n2_ref[...] = jnp.tanh(jnp.float32(ALPHA) * h2)


def _main_body(n1_ref, n2_ref, out_ref, *, br, n):
    i = pl.program_id(0)
    r0 = i * br
    n1b = n1_ref[pl.ds(r0, br), :]
    n2b = n2_ref[pl.ds(r0, br), :]
    dn = (((1,), (1,)), ((), ()))
    m1 = lax.dot_general(n1b, n2_ref[...], dn, preferred_element_type=jnp.float32)
    m2 = lax.dot_general(n2b, n1_ref[...], dn, preferred_element_type=jnp.float32)
    a_val = jnp.maximum(jnp.tanh(jnp.float32(ALPHA) * (m1 - m2)), jnp.float32(0.0))
    v = a_val + _noise_tile(r0, br, n)
    vb = lax.bitcast_convert_type(v, jnp.int32)

    def rowcount(mask_bool):
        return jnp.sum(mask_bool.astype(jnp.int32), axis=1, keepdims=True)

    one_bits = jnp.int32(_EXP_ONE)
    c1 = rowcount(vb >= one_bits)
    sat = c1 >= TOPK
    lo0 = jnp.where(sat, one_bits, 0)
    hi0 = jnp.where(sat, jnp.int32(_HI_INIT), one_bits - 1)
    clo0 = jnp.where(sat, c1, jnp.int32(n))
    chi0 = jnp.where(sat, 0, c1)

    def bs_cond(carry):
        lo, hi, _, _, _ = carry
        return jnp.any(lo < hi)

    def bs_body(carry):
        lo, hi, cl, ch, t = carry
        rng = hi - lo
        frac = (cl - TOPK).astype(jnp.float32) / jnp.maximum(
            (cl - ch).astype(jnp.float32), jnp.float32(1.0))
        step = (frac * rng.astype(jnp.float32)).astype(jnp.int32)
        interp = lo + jnp.clip(step, 1, rng)
        bisect = (lo + hi + 1) >> 1
        mid = jnp.where((t % 2) == 0, interp, bisect)
        mid = jnp.clip(mid, lo + 1, hi)
        cnt = rowcount(vb >= mid)
        ge = cnt >= TOPK
        lo = jnp.where(ge, mid, lo)
        cl = jnp.where(ge, cnt, cl)
        hi = jnp.where(ge, hi, mid - 1)
        ch = jnp.where(ge, ch, cnt)
        return lo, hi, cl, ch, t + 1

    tbits, _, _, _, _ = lax.while_loop(
        bs_cond, bs_body, (lo0, hi0, clo0, chi0, jnp.int32(0)))

    c_gt = rowcount(vb > tbits)
    need = TOPK - c_gt
    eq = vb == tbits
    e_cnt = rowcount(eq)
    col = lax.broadcasted_iota(jnp.int32, (br, n), 1)
    need_eff = jnp.where(e_cnt == need, 0, need)

    def tie_cond(carry):
        _, cnt = carry
        return jnp.any(cnt < need_eff)

    def tie_body(carry):
        last, cnt = carry
        cand = jnp.where(eq & (col > last), col, jnp.int32(n))
        m = jnp.min(cand, axis=1, keepdims=True)
        active = cnt < need_eff
        return jnp.where(active, m, last), cnt + active.astype(jnp.int32)

    last0 = jnp.full((br, 1), -1, jnp.int32)
    cnt0 = jnp.zeros((br, 1), jnp.int32)
    last, _ = lax.while_loop(tie_cond, tie_body, (last0, cnt0))
    cstar = jnp.where(need_eff == 0, jnp.int32(n - 1), last)

    mask = (vb > tbits) | (eq & (col <= cstar))
    out_ref[...] = jnp.where(mask, a_val, jnp.float32(0.0))


def kernel(idx, emb1, emb2, W1, b1, W2, b2):
    n, d = emb1.shape
    g1 = jnp.take(emb1, idx, axis=0)
    g2 = jnp.take(emb2, idx, axis=0)

    pr_block = min(n, 1024)
    n1, n2 = pl.pallas_call(
        _prologue_body,
        grid=(n // pr_block,),
        in_specs=[
            pl.BlockSpec((pr_block, d), lambda i: (i, 0)),
            pl.BlockSpec((pr_block, d), lambda i: (i, 0)),
            pl.BlockSpec((d, d), lambda i: (0, 0)),
            pl.BlockSpec((1, d), lambda i: (0, 0)),
            pl.BlockSpec((d, d), lambda i: (0, 0)),
            pl.BlockSpec((1, d), lambda i: (0, 0)),
        ],
        out_specs=[
            pl.BlockSpec((pr_block, d), lambda i: (i, 0)),
            pl.BlockSpec((pr_block, d), lambda i: (i, 0)),
        ],
        out_shape=[
            jax.ShapeDtypeStruct((n, d), jnp.float32),
            jax.ShapeDtypeStruct((n, d), jnp.float32),
        ],
        compiler_params=pltpu.CompilerParams(
            dimension_semantics=("parallel",),
        ),
    )(g1, g2, W1, b1.reshape(1, d), W2, b2.reshape(1, d))

    br = 128
    out = pl.pallas_call(
        functools.partial(_main_body, br=br, n=n),
        grid=(n // br,),
        in_specs=[
            pl.BlockSpec((n, d), lambda i: (0, 0)),
            pl.BlockSpec((n, d), lambda i: (0, 0)),
        ],
        out_specs=pl.BlockSpec((br, n), lambda i: (i, 0)),
        out_shape=jax.ShapeDtypeStruct((n, n), jnp.float32),
        compiler_params=pltpu.CompilerParams(
            dimension_semantics=("parallel",),
            vmem_limit_bytes=100 * 1024 * 1024,
        ),
    )(n1, n2)
    return out


# speedup vs baseline: 1.0778x; 1.0778x over previous
import functools

import jax
import jax.numpy as jnp
from jax import lax
from jax.experimental import pallas as pl
from jax.experimental.pallas import tpu as pltpu

ALPHA = 3.0
TOPK = 32

_EXP_ONE = 0x3F800000
_MANT_MASK = 0x007FFFFF
_HI_INIT = 0x3F814800


def _rotl(x, r):
    return (x << r) | ((x >> (32 - r)) & ((1 << r) - 1))


def _threefry2x32_zero(x1):
    k0 = jnp.int32(0)
    k1 = jnp.int32(42)
    k2 = k0 ^ k1 ^ jnp.int32(0x1BD11BDA)
    ks = (k0, k1, k2)
    rotations = ((13, 15, 26, 6), (17, 29, 16, 24))
    x1 = x1 + k1
    x0 = x1
    x1 = x0 ^ _rotl(x1, 13)
    first = True
    for i in range(5):
        for r in rotations[i % 2]:
            if first:
                first = False
                continue
            x0 = x0 + x1
            x1 = _rotl(x1, r)
            x1 = x0 ^ x1
        x0 = x0 + ks[(i + 1) % 3]
        x1 = x1 + ks[(i + 2) % 3] + jnp.int32(i + 1)
    return x0, x1


def _noise_tile(r0, br, n):
    row = r0 + lax.broadcasted_iota(jnp.int32, (br, n), 0)
    col = lax.broadcasted_iota(jnp.int32, (br, n), 1)
    f = row * n + col
    o0, o1 = _threefry2x32_zero(f)
    bits = o0 ^ o1
    fb = ((bits >> 9) & _MANT_MASK) | jnp.int32(_EXP_ONE)
    u = lax.bitcast_convert_type(fb, jnp.float32) - jnp.float32(1.0)
    return u * jnp.float32(0.01)


def _prologue_body(x1_ref, x2_ref, w1_ref, b1_ref, w2_ref, b2_ref, n1_ref, n2_ref):
    dn = (((1,), (1,)), ((), ()))
    h1 = lax.dot_general(x1_ref[...], w1_ref[...], dn,
                         preferred_element_type=jnp.float32) + b1_ref[...]
    n1_ref[...] = jnp.tanh(jnp.float32(ALPHA) * h1)
    h2 = lax.dot_general(x2_ref[...], w2_ref[...], dn,
                         preferred_element_type=jnp.float32) + b2_ref[...]
    n2_ref[...] = jnp.tanh(jnp.float32(ALPHA) * h2)


def _main_body(n1_ref, n2_ref, out_ref, *, br, n):
    i = pl.program_id(0)
    r0 = i * br
    n1b = n1_ref[pl.ds(r0, br), :]
    n2b = n2_ref[pl.ds(r0, br), :]
    dn = (((1,), (1,)), ((), ()))
    m1 = lax.dot_general(n1b, n2_ref[...], dn, preferred_element_type=jnp.float32)
    m2 = lax.dot_general(n2b, n1_ref[...], dn, preferred_element_type=jnp.float32)
    a_val = jnp.maximum(jnp.tanh(jnp.float32(ALPHA) * (m1 - m2)), jnp.float32(0.0))
    v = a_val + _noise_tile(r0, br, n)
    vb = lax.bitcast_convert_type(v, jnp.int32)

    def rowcount(mask_bool):
        return jnp.sum(mask_bool.astype(jnp.int32), axis=1, keepdims=True)

    one_bits = jnp.int32(_EXP_ONE)
    c1 = rowcount(vb >= one_bits)
    sat = c1 >= TOPK
    lo0 = jnp.where(sat, one_bits, 0)
    hi0 = jnp.where(sat, jnp.int32(_HI_INIT), one_bits - 1)

    def bs_cond(carry):
        lo, hi = carry
        return jnp.any(lo < hi)

    def bs_step(lo, hi):
        mid = (lo + hi + 1) >> 1
        ge = rowcount(vb >= mid) >= TOPK
        return jnp.where(ge, mid, lo), jnp.where(ge, hi, mid - 1)

    def bs_body(carry):
        lo, hi = carry
        lo, hi = bs_step(lo, hi)
        return bs_step(lo, hi)

    tbits, _ = lax.while_loop(bs_cond, bs_body, (lo0, hi0))

    c_gt = rowcount(vb > tbits)
    need = TOPK - c_gt
    eq = vb == tbits
    e_cnt = rowcount(eq)
    col = lax.broadcasted_iota(jnp.int32, (br, n), 1)
    need_eff = jnp.where(e_cnt == need, 0, need)

    def tie_cond(carry):
        _, cnt = carry
        return jnp.any(cnt < need_eff)

    def tie_body(carry):
        last, cnt = carry
        cand = jnp.where(eq & (col > last), col, jnp.int32(n))
        m = jnp.min(cand, axis=1, keepdims=True)
        active = cnt < need_eff
        return jnp.where(active, m, last), cnt + active.astype(jnp.int32)

    last0 = jnp.full((br, 1), -1, jnp.int32)
    cnt0 = jnp.zeros((br, 1), jnp.int32)
    last, _ = lax.while_loop(tie_cond, tie_body, (last0, cnt0))
    cstar = jnp.where(need_eff == 0, jnp.int32(n - 1), last)

    mask = (vb > tbits) | (eq & (col <= cstar))
    out_ref[...] = jnp.where(mask, a_val, jnp.float32(0.0))


def kernel(idx, emb1, emb2, W1, b1, W2, b2):
    n, d = emb1.shape
    g1 = jnp.take(emb1, idx, axis=0)
    g2 = jnp.take(emb2, idx, axis=0)

    pr_block = min(n, 1024)
    n1, n2 = pl.pallas_call(
        _prologue_body,
        grid=(n // pr_block,),
        in_specs=[
            pl.BlockSpec((pr_block, d), lambda i: (i, 0)),
            pl.BlockSpec((pr_block, d), lambda i: (i, 0)),
            pl.BlockSpec((d, d), lambda i: (0, 0)),
            pl.BlockSpec((1, d), lambda i: (0, 0)),
            pl.BlockSpec((d, d), lambda i: (0, 0)),
            pl.BlockSpec((1, d), lambda i: (0, 0)),
        ],
        out_specs=[
            pl.BlockSpec((pr_block, d), lambda i: (i, 0)),
            pl.BlockSpec((pr_block, d), lambda i: (i, 0)),
        ],
        out_shape=[
            jax.ShapeDtypeStruct((n, d), jnp.float32),
            jax.ShapeDtypeStruct((n, d), jnp.float32),
        ],
        compiler_params=pltpu.CompilerParams(
            dimension_semantics=("parallel",),
        ),
    )(g1, g2, W1, b1.reshape(1, d), W2, b2.reshape(1, d))

    br = 128
    out = pl.pallas_call(
        functools.partial(_main_body, br=br, n=n),
        grid=(n // br,),
        in_specs=[
            pl.BlockSpec((n, d), lambda i: (0, 0)),
            pl.BlockSpec((n, d), lambda i: (0, 0)),
        ],
        out_specs=pl.BlockSpec((br, n), lambda i: (i, 0)),
        out_shape=jax.ShapeDtypeStruct((n, n), jnp.float32),
        compiler_params=pltpu.CompilerParams(
            dimension_semantics=("parallel",),
            vmem_limit_bytes=100 * 1024 * 1024,
        ),
    )(n1, n2)
    return out


# speedup vs baseline: 1.0806x; 1.0025x over previous
import functools

import jax
import jax.numpy as jnp
from jax import lax
from jax.experimental import pallas as pl
from jax.experimental.pallas import tpu as pltpu
from jax.experimental.pallas import tpu_sc as plsc

ALPHA = 3.0
TOPK = 32

_EXP_ONE = 0x3F800000
_MANT_MASK = 0x007FFFFF
_HI_INIT = 0x3F814800


def _rotl(x, r):
    return (x << r) | ((x >> (32 - r)) & ((1 << r) - 1))


def _threefry2x32_zero(x1):
    k0 = jnp.int32(0)
    k1 = jnp.int32(42)
    k2 = k0 ^ k1 ^ jnp.int32(0x1BD11BDA)
    ks = (k0, k1, k2)
    rotations = ((13, 15, 26, 6), (17, 29, 16, 24))
    x1 = x1 + k1
    x0 = x1
    x1 = x0 ^ _rotl(x1, 13)
    first = True
    for i in range(5):
        for r in rotations[i % 2]:
            if first:
                first = False
                continue
            x0 = x0 + x1
            x1 = _rotl(x1, r)
            x1 = x0 ^ x1
        x0 = x0 + ks[(i + 1) % 3]
        x1 = x1 + ks[(i + 2) % 3] + jnp.int32(i + 1)
    return x0, x1


def _noise_tile(r0, br, n):
    row = r0 + lax.broadcasted_iota(jnp.int32, (br, n), 0)
    col = lax.broadcasted_iota(jnp.int32, (br, n), 1)
    f = row * n + col
    o0, o1 = _threefry2x32_zero(f)
    bits = o0 ^ o1
    fb = ((bits >> 9) & _MANT_MASK) | jnp.int32(_EXP_ONE)
    u = lax.bitcast_convert_type(fb, jnp.float32) - jnp.float32(1.0)
    return u * jnp.float32(0.01)


_SC_CORES = 2
_SC_SUBCORES = 16
_SC_CHUNK = 128


def _sc_gather_pair(emb1, emb2, idx):
    n, d = emb1.shape
    nw = _SC_CORES * _SC_SUBCORES
    b_per_w = n // nw
    ch = min(_SC_CHUNK, b_per_w)
    mesh = plsc.VectorSubcoreMesh(core_axis_name="c", subcore_axis_name="s")

    def body(e1_hbm, e2_hbm, idx_hbm, o1_hbm, o2_hbm, idx_v, rows_v, sem):
        wid = lax.axis_index("s") * _SC_CORES + lax.axis_index("c")
        base = wid * b_per_w
        for j in range(b_per_w // ch):
            off = base + j * ch
            pltpu.sync_copy(idx_hbm.at[pl.ds(off, ch)], idx_v)
            pltpu.async_copy(e1_hbm.at[idx_v], rows_v, sem).wait()
            pltpu.sync_copy(rows_v, o1_hbm.at[pl.ds(off, ch)])
            pltpu.async_copy(e2_hbm.at[idx_v], rows_v, sem).wait()
            pltpu.sync_copy(rows_v, o2_hbm.at[pl.ds(off, ch)])

    fn = pl.kernel(
        body,
        out_type=[
            jax.ShapeDtypeStruct((n, d), jnp.float32),
            jax.ShapeDtypeStruct((n, d), jnp.float32),
        ],
        mesh=mesh,
        scratch_types=[
            pltpu.VMEM((ch,), jnp.int32),
            pltpu.VMEM((ch, d), jnp.float32),
            pltpu.SemaphoreType.DMA,
        ],
    )
    return fn(emb1, emb2, idx)


def _prologue_body(x1_ref, x2_ref, w1_ref, b1_ref, w2_ref, b2_ref, n1_ref, n2_ref):
    dn = (((1,), (1,)), ((), ()))
    h1 = lax.dot_general(x1_ref[...], w1_ref[...], dn,
                         preferred_element_type=jnp.float32) + b1_ref[...]
    n1_ref[...] = jnp.tanh(jnp.float32(ALPHA) * h1)
    h2 = lax.dot_general(x2_ref[...], w2_ref[...], dn,
                         preferred_element_type=jnp.float32) + b2_ref[...]
    n2_ref[...] = jnp.tanh(jnp.float32(ALPHA) * h2)


def _main_body(n1_ref, n2_ref, out_ref, *, br, n):
    i = pl.program_id(0)
    r0 = i * br
    n1b = n1_ref[pl.ds(r0, br), :]
    n2b = n2_ref[pl.ds(r0, br), :]
    dn = (((1,), (1,)), ((), ()))
    m1 = lax.dot_general(n1b, n2_ref[...], dn, preferred_element_type=jnp.float32)
    m2 = lax.dot_general(n2b, n1_ref[...], dn, preferred_element_type=jnp.float32)
    a_val = jnp.maximum(jnp.tanh(jnp.float32(ALPHA) * (m1 - m2)), jnp.float32(0.0))
    v = a_val + _noise_tile(r0, br, n)
    vb = lax.bitcast_convert_type(v, jnp.int32)

    def rowcount(mask_bool):
        return jnp.sum(mask_bool.astype(jnp.int32), axis=1, keepdims=True)

    one_bits = jnp.int32(_EXP_ONE)
    c1 = rowcount(vb >= one_bits)
    sat = c1 >= TOPK
    lo0 = jnp.where(sat, one_bits, 0)
    hi0 = jnp.where(sat, jnp.int32(_HI_INIT), one_bits - 1)

    def bs_cond(carry):
        lo, hi = carry
        return jnp.any(lo < hi)

    def bs_step(lo, hi):
        mid = (lo + hi + 1) >> 1
        ge = rowcount(vb >= mid) >= TOPK
        return jnp.where(ge, mid, lo), jnp.where(ge, hi, mid - 1)

    def bs_body(carry):
        lo, hi = carry
        lo, hi = bs_step(lo, hi)
        return bs_step(lo, hi)

    tbits, _ = lax.while_loop(bs_cond, bs_body, (lo0, hi0))

    c_gt = rowcount(vb > tbits)
    need = TOPK - c_gt
    eq = vb == tbits
    e_cnt = rowcount(eq)
    col = lax.broadcasted_iota(jnp.int32, (br, n), 1)
    need_eff = jnp.where(e_cnt == need, 0, need)

    def tie_cond(carry):
        _, cnt = carry
        return jnp.any(cnt < need_eff)

    def tie_body(carry):
        last, cnt = carry
        cand = jnp.where(eq & (col > last), col, jnp.int32(n))
        m = jnp.min(cand, axis=1, keepdims=True)
        active = cnt < need_eff
        return jnp.where(active, m, last), cnt + active.astype(jnp.int32)

    last0 = jnp.full((br, 1), -1, jnp.int32)
    cnt0 = jnp.zeros((br, 1), jnp.int32)
    last, _ = lax.while_loop(tie_cond, tie_body, (last0, cnt0))
    cstar = jnp.where(need_eff == 0, jnp.int32(n - 1), last)

    mask = (vb > tbits) | (eq & (col <= cstar))
    out_ref[...] = jnp.where(mask, a_val, jnp.float32(0.0))


def kernel(idx, emb1, emb2, W1, b1, W2, b2):
    n, d = emb1.shape
    g1, g2 = _sc_gather_pair(emb1, emb2, idx)

    pr_block = min(n, 1024)
    n1, n2 = pl.pallas_call(
        _prologue_body,
        grid=(n // pr_block,),
        in_specs=[
            pl.BlockSpec((pr_block, d), lambda i: (i, 0)),
            pl.BlockSpec((pr_block, d), lambda i: (i, 0)),
            pl.BlockSpec((d, d), lambda i: (0, 0)),
            pl.BlockSpec((1, d), lambda i: (0, 0)),
            pl.BlockSpec((d, d), lambda i: (0, 0)),
            pl.BlockSpec((1, d), lambda i: (0, 0)),
        ],
        out_specs=[
            pl.BlockSpec((pr_block, d), lambda i: (i, 0)),
            pl.BlockSpec((pr_block, d), lambda i: (i, 0)),
        ],
        out_shape=[
            jax.ShapeDtypeStruct((n, d), jnp.float32),
            jax.ShapeDtypeStruct((n, d), jnp.float32),
        ],
        compiler_params=pltpu.CompilerParams(
            dimension_semantics=("parallel",),
        ),
    )(g1, g2, W1, b1.reshape(1, d), W2, b2.reshape(1, d))

    br = 128
    out = pl.pallas_call(
        functools.partial(_main_body, br=br, n=n),
        grid=(n // br,),
        in_specs=[
            pl.BlockSpec((n, d), lambda i: (0, 0)),
            pl.BlockSpec((n, d), lambda i: (0, 0)),
        ],
        out_specs=pl.BlockSpec((br, n), lambda i: (i, 0)),
        out_shape=jax.ShapeDtypeStruct((n, n), jnp.float32),
        compiler_params=pltpu.CompilerParams(
            dimension_semantics=("parallel",),
            vmem_limit_bytes=100 * 1024 * 1024,
        ),
    )(n1, n2)
    return out


# speedup vs baseline: 1.1126x; 1.0296x over previous
import functools

import jax
import jax.numpy as jnp
from jax import lax
from jax.experimental import pallas as pl
from jax.experimental.pallas import tpu as pltpu
from jax.experimental.pallas import tpu_sc as plsc

ALPHA = 3.0
TOPK = 32

_EXP_ONE = 0x3F800000
_MANT_MASK = 0x007FFFFF
_HI_INIT = 0x3F814800


def _rotl(x, r):
    return (x << r) | ((x >> (32 - r)) & ((1 << r) - 1))


def _threefry2x32_zero(x1):
    k0 = jnp.int32(0)
    k1 = jnp.int32(42)
    k2 = k0 ^ k1 ^ jnp.int32(0x1BD11BDA)
    ks = (k0, k1, k2)
    rotations = ((13, 15, 26, 6), (17, 29, 16, 24))
    x1 = x1 + k1
    x0 = x1
    x1 = x0 ^ _rotl(x1, 13)
    first = True
    for i in range(5):
        for r in rotations[i % 2]:
            if first:
                first = False
                continue
            x0 = x0 + x1
            x1 = _rotl(x1, r)
            x1 = x0 ^ x1
        x0 = x0 + ks[(i + 1) % 3]
        x1 = x1 + ks[(i + 2) % 3] + jnp.int32(i + 1)
    return x0, x1


def _noise_tile(r0, br, n):
    row = r0 + lax.broadcasted_iota(jnp.int32, (br, n), 0)
    col = lax.broadcasted_iota(jnp.int32, (br, n), 1)
    f = row * n + col
    o0, o1 = _threefry2x32_zero(f)
    bits = o0 ^ o1
    fb = ((bits >> 9) & _MANT_MASK) | jnp.int32(_EXP_ONE)
    u = lax.bitcast_convert_type(fb, jnp.float32) - jnp.float32(1.0)
    return u * jnp.float32(0.01)


_SC_CORES = 2
_SC_SUBCORES = 16
_SC_CHUNK = 128


def _sc_gather_pair(emb1, emb2, idx):
    n, d = emb1.shape
    nw = _SC_CORES * _SC_SUBCORES
    b_per_w = n // nw
    ch = min(_SC_CHUNK, b_per_w)
    mesh = plsc.VectorSubcoreMesh(core_axis_name="c", subcore_axis_name="s")

    def body(e1_hbm, e2_hbm, idx_hbm, o1_hbm, o2_hbm, idx_v, rows_v, sem):
        wid = lax.axis_index("s") * _SC_CORES + lax.axis_index("c")
        base = wid * b_per_w
        for j in range(b_per_w // ch):
            off = base + j * ch
            pltpu.sync_copy(idx_hbm.at[pl.ds(off, ch)], idx_v)
            pltpu.async_copy(e1_hbm.at[idx_v], rows_v, sem).wait()
            pltpu.sync_copy(rows_v, o1_hbm.at[pl.ds(off, ch)])
            pltpu.async_copy(e2_hbm.at[idx_v], rows_v, sem).wait()
            pltpu.sync_copy(rows_v, o2_hbm.at[pl.ds(off, ch)])

    fn = pl.kernel(
        body,
        out_type=[
            jax.ShapeDtypeStruct((n, d), jnp.float32),
            jax.ShapeDtypeStruct((n, d), jnp.float32),
        ],
        mesh=mesh,
        scratch_types=[
            pltpu.VMEM((ch,), jnp.int32),
            pltpu.VMEM((ch, d), jnp.float32),
            pltpu.SemaphoreType.DMA,
        ],
    )
    return fn(emb1, emb2, idx)


def _prologue_body(x1_ref, x2_ref, w1_ref, b1_ref, w2_ref, b2_ref, n1_ref, n2_ref):
    dn = (((1,), (1,)), ((), ()))
    h1 = lax.dot_general(x1_ref[...], w1_ref[...], dn,
                         preferred_element_type=jnp.float32) + b1_ref[...]
    n1_ref[...] = jnp.tanh(jnp.float32(ALPHA) * h1)
    h2 = lax.dot_general(x2_ref[...], w2_ref[...], dn,
                         preferred_element_type=jnp.float32) + b2_ref[...]
    n2_ref[...] = jnp.tanh(jnp.float32(ALPHA) * h2)


def _main_body(n1_ref, n2_ref, out_ref, *, br, n):
    i = pl.program_id(0)
    r0 = i * br
    n1b = n1_ref[pl.ds(r0, br), :]
    n2b = n2_ref[pl.ds(r0, br), :]
    dn = (((1,), (1,)), ((), ()))
    m1 = lax.dot_general(n1b, n2_ref[...], dn, preferred_element_type=jnp.float32)
    m2 = lax.dot_general(n2b, n1_ref[...], dn, preferred_element_type=jnp.float32)
    a_val = jnp.maximum(jnp.tanh(jnp.float32(ALPHA) * (m1 - m2)), jnp.float32(0.0))
    v = a_val + _noise_tile(r0, br, n)
    vb = lax.bitcast_convert_type(v, jnp.int32)

    def rowcount(mask_bool):
        return jnp.sum(mask_bool.astype(jnp.int32), axis=1, keepdims=True)

    one_bits = jnp.int32(_EXP_ONE)
    c1 = rowcount(vb >= one_bits)
    sat = c1 >= TOPK
    lo0 = jnp.where(sat, one_bits, 0)
    hi0 = jnp.where(sat, jnp.int32(_HI_INIT), one_bits - 1)

    def bs_step(lo, hi):
        mid = (lo + hi + 1) >> 1
        ge = rowcount(vb >= mid) >= TOPK
        return jnp.where(ge, mid, lo), jnp.where(ge, hi, mid - 1)

    def bs_body17(_, carry):
        return bs_step(*carry)

    lo1, hi1 = lax.fori_loop(0, 17, bs_body17, (lo0, hi0))

    def bs_cond(carry):
        lo, hi = carry
        return jnp.any(lo < hi)

    def bs_body(carry):
        return bs_step(*carry)

    tbits, _ = lax.while_loop(bs_cond, bs_body, (lo1, hi1))

    c_gt = rowcount(vb > tbits)
    need = TOPK - c_gt
    eq = vb == tbits
    e_cnt = rowcount(eq)
    col = lax.broadcasted_iota(jnp.int32, (br, n), 1)
    need_eff = jnp.where(e_cnt == need, 0, need)

    def tie_cond(carry):
        _, cnt = carry
        return jnp.any(cnt < need_eff)

    def tie_body(carry):
        last, cnt = carry
        cand = jnp.where(eq & (col > last), col, jnp.int32(n))
        m = jnp.min(cand, axis=1, keepdims=True)
        active = cnt < need_eff
        return jnp.where(active, m, last), cnt + active.astype(jnp.int32)

    last0 = jnp.full((br, 1), -1, jnp.int32)
    cnt0 = jnp.zeros((br, 1), jnp.int32)
    last, _ = lax.while_loop(tie_cond, tie_body, (last0, cnt0))
    cstar = jnp.where(need_eff == 0, jnp.int32(n - 1), last)

    mask = (vb > tbits) | (eq & (col <= cstar))
    out_ref[...] = jnp.where(mask, a_val, jnp.float32(0.0))


def kernel(idx, emb1, emb2, W1, b1, W2, b2):
    n, d = emb1.shape
    g1, g2 = _sc_gather_pair(emb1, emb2, idx)

    pr_block = min(n, 1024)
    n1, n2 = pl.pallas_call(
        _prologue_body,
        grid=(n // pr_block,),
        in_specs=[
            pl.BlockSpec((pr_block, d), lambda i: (i, 0)),
            pl.BlockSpec((pr_block, d), lambda i: (i, 0)),
            pl.BlockSpec((d, d), lambda i: (0, 0)),
            pl.BlockSpec((1, d), lambda i: (0, 0)),
            pl.BlockSpec((d, d), lambda i: (0, 0)),
            pl.BlockSpec((1, d), lambda i: (0, 0)),
        ],
        out_specs=[
            pl.BlockSpec((pr_block, d), lambda i: (i, 0)),
            pl.BlockSpec((pr_block, d), lambda i: (i, 0)),
        ],
        out_shape=[
            jax.ShapeDtypeStruct((n, d), jnp.float32),
            jax.ShapeDtypeStruct((n, d), jnp.float32),
        ],
        compiler_params=pltpu.CompilerParams(
            dimension_semantics=("parallel",),
        ),
    )(g1, g2, W1, b1.reshape(1, d), W2, b2.reshape(1, d))

    br = 128
    out = pl.pallas_call(
        functools.partial(_main_body, br=br, n=n),
        grid=(n // br,),
        in_specs=[
            pl.BlockSpec((n, d), lambda i: (0, 0)),
            pl.BlockSpec((n, d), lambda i: (0, 0)),
        ],
        out_specs=pl.BlockSpec((br, n), lambda i: (i, 0)),
        out_shape=jax.ShapeDtypeStruct((n, n), jnp.float32),
        compiler_params=pltpu.CompilerParams(
            dimension_semantics=("parallel",),
            vmem_limit_bytes=100 * 1024 * 1024,
        ),
    )(n1, n2)
    return out


# speedup vs baseline: 1.1660x; 1.0480x over previous
import functools

import jax
import jax.numpy as jnp
from jax import lax
from jax.experimental import pallas as pl
from jax.experimental.pallas import tpu as pltpu
from jax.experimental.pallas import tpu_sc as plsc

ALPHA = 3.0
TOPK = 32

_EXP_ONE = 0x3F800000
_MANT_MASK = 0x007FFFFF
_HI_INIT = 0x3F814800


def _rotl(x, r):
    return (x << r) | ((x >> (32 - r)) & ((1 << r) - 1))


def _threefry2x32_zero(x1):
    k0 = jnp.int32(0)
    k1 = jnp.int32(42)
    k2 = k0 ^ k1 ^ jnp.int32(0x1BD11BDA)
    ks = (k0, k1, k2)
    rotations = ((13, 15, 26, 6), (17, 29, 16, 24))
    x1 = x1 + k1
    x0 = x1
    x1 = x0 ^ _rotl(x1, 13)
    first = True
    for i in range(5):
        for r in rotations[i % 2]:
            if first:
                first = False
                continue
            x0 = x0 + x1
            x1 = _rotl(x1, r)
            x1 = x0 ^ x1
        x0 = x0 + ks[(i + 1) % 3]
        x1 = x1 + ks[(i + 2) % 3] + jnp.int32(i + 1)
    return x0, x1


def _noise_tile(r0, br, n):
    row = r0 + lax.broadcasted_iota(jnp.int32, (br, n), 0)
    col = lax.broadcasted_iota(jnp.int32, (br, n), 1)
    f = row * n + col
    o0, o1 = _threefry2x32_zero(f)
    bits = o0 ^ o1
    fb = ((bits >> 9) & _MANT_MASK) | jnp.int32(_EXP_ONE)
    u = lax.bitcast_convert_type(fb, jnp.float32) - jnp.float32(1.0)
    return u * jnp.float32(0.01)


_SC_CORES = 2
_SC_SUBCORES = 16
_SC_CHUNK = 128


def _sc_gather_pair(emb1, emb2, idx):
    n, d = emb1.shape
    nw = _SC_CORES * _SC_SUBCORES
    b_per_w = n // nw
    ch = min(_SC_CHUNK, b_per_w)
    mesh = plsc.VectorSubcoreMesh(core_axis_name="c", subcore_axis_name="s")

    def body(e1_hbm, e2_hbm, idx_hbm, o1_hbm, o2_hbm, idx_v, rows_v, sem):
        wid = lax.axis_index("s") * _SC_CORES + lax.axis_index("c")
        base = wid * b_per_w
        for j in range(b_per_w // ch):
            off = base + j * ch
            pltpu.sync_copy(idx_hbm.at[pl.ds(off, ch)], idx_v)
            pltpu.async_copy(e1_hbm.at[idx_v], rows_v, sem).wait()
            pltpu.sync_copy(rows_v, o1_hbm.at[pl.ds(off, ch)])
            pltpu.async_copy(e2_hbm.at[idx_v], rows_v, sem).wait()
            pltpu.sync_copy(rows_v, o2_hbm.at[pl.ds(off, ch)])

    fn = pl.kernel(
        body,
        out_type=[
            jax.ShapeDtypeStruct((n, d), jnp.float32),
            jax.ShapeDtypeStruct((n, d), jnp.float32),
        ],
        mesh=mesh,
        scratch_types=[
            pltpu.VMEM((ch,), jnp.int32),
            pltpu.VMEM((ch, d), jnp.float32),
            pltpu.SemaphoreType.DMA,
        ],
    )
    return fn(emb1, emb2, idx)


def _prologue_body(x1_ref, x2_ref, w1_ref, b1_ref, w2_ref, b2_ref, n1_ref, n2_ref):
    dn = (((1,), (1,)), ((), ()))
    h1 = lax.dot_general(x1_ref[...], w1_ref[...], dn,
                         preferred_element_type=jnp.float32) + b1_ref[...]
    n1_ref[...] = jnp.tanh(jnp.float32(ALPHA) * h1)
    h2 = lax.dot_general(x2_ref[...], w2_ref[...], dn,
                         preferred_element_type=jnp.float32) + b2_ref[...]
    n2_ref[...] = jnp.tanh(jnp.float32(ALPHA) * h2)


def _main_body(n1_ref, n2_ref, out_ref, *, br, n):
    i = pl.program_id(0)
    r0 = i * br
    n1b = n1_ref[pl.ds(r0, br), :]
    n2b = n2_ref[pl.ds(r0, br), :]
    dn = (((1,), (1,)), ((), ()))
    m1 = lax.dot_general(n1b, n2_ref[...], dn, preferred_element_type=jnp.float32)
    m2 = lax.dot_general(n2b, n1_ref[...], dn, preferred_element_type=jnp.float32)
    a_val = jnp.maximum(jnp.tanh(jnp.float32(ALPHA) * (m1 - m2)), jnp.float32(0.0))
    v = a_val + _noise_tile(r0, br, n)
    vb = lax.bitcast_convert_type(v, jnp.int32)

    def rowcount(mask_bool):
        return jnp.sum(mask_bool.astype(jnp.int32), axis=1, keepdims=True)

    one_bits = jnp.int32(_EXP_ONE)
    c1 = rowcount(vb >= one_bits)
    sat = c1 >= TOPK
    lo0 = jnp.where(sat, one_bits, 0)
    hi0 = jnp.where(sat, jnp.int32(_HI_INIT), one_bits - 1)

    def bs_step(lo, hi):
        mid = (lo + hi + 1) >> 1
        ge = rowcount(vb >= mid) >= TOPK
        return jnp.where(ge, mid, lo), jnp.where(ge, hi, mid - 1)

    def bs_body17(_, carry):
        return bs_step(*carry)

    lo1, hi1 = lax.fori_loop(0, 17, bs_body17, (lo0, hi0), unroll=True)

    def bs_cond(carry):
        lo, hi = carry
        return jnp.any(lo < hi)

    def bs_body(carry):
        return bs_step(*carry)

    tbits, _ = lax.while_loop(bs_cond, bs_body, (lo1, hi1))

    c_gt = rowcount(vb > tbits)
    need = TOPK - c_gt
    eq = vb == tbits
    e_cnt = rowcount(eq)
    col = lax.broadcasted_iota(jnp.int32, (br, n), 1)
    need_eff = jnp.where(e_cnt == need, 0, need)

    def tie_cond(carry):
        _, cnt = carry
        return jnp.any(cnt < need_eff)

    def tie_body(carry):
        last, cnt = carry
        cand = jnp.where(eq & (col > last), col, jnp.int32(n))
        m = jnp.min(cand, axis=1, keepdims=True)
        active = cnt < need_eff
        return jnp.where(active, m, last), cnt + active.astype(jnp.int32)

    last0 = jnp.full((br, 1), -1, jnp.int32)
    cnt0 = jnp.zeros((br, 1), jnp.int32)
    last, _ = lax.while_loop(tie_cond, tie_body, (last0, cnt0))
    cstar = jnp.where(need_eff == 0, jnp.int32(n - 1), last)

    mask = (vb > tbits) | (eq & (col <= cstar))
    out_ref[...] = jnp.where(mask, a_val, jnp.float32(0.0))


def kernel(idx, emb1, emb2, W1, b1, W2, b2):
    n, d = emb1.shape
    g1, g2 = _sc_gather_pair(emb1, emb2, idx)

    pr_block = min(n, 1024)
    n1, n2 = pl.pallas_call(
        _prologue_body,
        grid=(n // pr_block,),
        in_specs=[
            pl.BlockSpec((pr_block, d), lambda i: (i, 0)),
            pl.BlockSpec((pr_block, d), lambda i: (i, 0)),
            pl.BlockSpec((d, d), lambda i: (0, 0)),
            pl.BlockSpec((1, d), lambda i: (0, 0)),
            pl.BlockSpec((d, d), lambda i: (0, 0)),
            pl.BlockSpec((1, d), lambda i: (0, 0)),
        ],
        out_specs=[
            pl.BlockSpec((pr_block, d), lambda i: (i, 0)),
            pl.BlockSpec((pr_block, d), lambda i: (i, 0)),
        ],
        out_shape=[
            jax.ShapeDtypeStruct((n, d), jnp.float32),
            jax.ShapeDtypeStruct((n, d), jnp.float32),
        ],
        compiler_params=pltpu.CompilerParams(
            dimension_semantics=("parallel",),
        ),
    )(g1, g2, W1, b1.reshape(1, d), W2, b2.reshape(1, d))

    br = 128
    out = pl.pallas_call(
        functools.partial(_main_body, br=br, n=n),
        grid=(n // br,),
        in_specs=[
            pl.BlockSpec((n, d), lambda i: (0, 0)),
            pl.BlockSpec((n, d), lambda i: (0, 0)),
        ],
        out_specs=pl.BlockSpec((br, n), lambda i: (i, 0)),
        out_shape=jax.ShapeDtypeStruct((n, n), jnp.float32),
        compiler_params=pltpu.CompilerParams(
            dimension_semantics=("parallel",),
            vmem_limit_bytes=100 * 1024 * 1024,
        ),
    )(n1, n2)
    return out


# speedup vs baseline: 1.1831x; 1.0147x over previous
import functools

import jax
import jax.numpy as jnp
from jax import lax
from jax.experimental import pallas as pl
from jax.experimental.pallas import tpu as pltpu
from jax.experimental.pallas import tpu_sc as plsc

ALPHA = 3.0
TOPK = 32

_EXP_ONE = 0x3F800000
_MANT_MASK = 0x007FFFFF
_HI_INIT = 0x3F814800


def _rotl(x, r):
    return (x << r) | ((x >> (32 - r)) & ((1 << r) - 1))


def _threefry2x32_zero(x1):
    k0 = jnp.int32(0)
    k1 = jnp.int32(42)
    k2 = k0 ^ k1 ^ jnp.int32(0x1BD11BDA)
    ks = (k0, k1, k2)
    rotations = ((13, 15, 26, 6), (17, 29, 16, 24))
    x1 = x1 + k1
    x0 = x1
    x1 = x0 ^ _rotl(x1, 13)
    first = True
    for i in range(5):
        for r in rotations[i % 2]:
            if first:
                first = False
                continue
            x0 = x0 + x1
            x1 = _rotl(x1, r)
            x1 = x0 ^ x1
        x0 = x0 + ks[(i + 1) % 3]
        x1 = x1 + ks[(i + 2) % 3] + jnp.int32(i + 1)
    return x0, x1


def _noise_tile(r0, br, n):
    row = r0 + lax.broadcasted_iota(jnp.int32, (br, n), 0)
    col = lax.broadcasted_iota(jnp.int32, (br, n), 1)
    f = row * n + col
    o0, o1 = _threefry2x32_zero(f)
    bits = o0 ^ o1
    fb = ((bits >> 9) & _MANT_MASK) | jnp.int32(_EXP_ONE)
    u = lax.bitcast_convert_type(fb, jnp.float32) - jnp.float32(1.0)
    return u * jnp.float32(0.01)


_SC_CORES = 2
_SC_SUBCORES = 16
_SC_CHUNK = 128


def _sc_gather_pair(emb1, emb2, idx):
    n, d = emb1.shape
    nw = _SC_CORES * _SC_SUBCORES
    b_per_w = n // nw
    ch = min(_SC_CHUNK, b_per_w)
    mesh = plsc.VectorSubcoreMesh(core_axis_name="c", subcore_axis_name="s")

    def body(e1_hbm, e2_hbm, idx_hbm, o1_hbm, o2_hbm, idx_v, rows_v, sem):
        wid = lax.axis_index("s") * _SC_CORES + lax.axis_index("c")
        base = wid * b_per_w
        for j in range(b_per_w // ch):
            off = base + j * ch
            pltpu.sync_copy(idx_hbm.at[pl.ds(off, ch)], idx_v)
            pltpu.async_copy(e1_hbm.at[idx_v], rows_v, sem).wait()
            pltpu.sync_copy(rows_v, o1_hbm.at[pl.ds(off, ch)])
            pltpu.async_copy(e2_hbm.at[idx_v], rows_v, sem).wait()
            pltpu.sync_copy(rows_v, o2_hbm.at[pl.ds(off, ch)])

    fn = pl.kernel(
        body,
        out_type=[
            jax.ShapeDtypeStruct((n, d), jnp.float32),
            jax.ShapeDtypeStruct((n, d), jnp.float32),
        ],
        mesh=mesh,
        scratch_types=[
            pltpu.VMEM((ch,), jnp.int32),
            pltpu.VMEM((ch, d), jnp.float32),
            pltpu.SemaphoreType.DMA,
        ],
    )
    return fn(emb1, emb2, idx)


def _prologue_body(x1_ref, x2_ref, w1_ref, b1_ref, w2_ref, b2_ref, n1_ref, n2_ref):
    dn = (((1,), (1,)), ((), ()))
    h1 = lax.dot_general(x1_ref[...], w1_ref[...], dn,
                         preferred_element_type=jnp.float32) + b1_ref[...]
    n1_ref[...] = jnp.tanh(jnp.float32(ALPHA) * h1)
    h2 = lax.dot_general(x2_ref[...], w2_ref[...], dn,
                         preferred_element_type=jnp.float32) + b2_ref[...]
    n2_ref[...] = jnp.tanh(jnp.float32(ALPHA) * h2)


def _main_body(n1_ref, n2_ref, out_ref, *, br, n):
    i = pl.program_id(0)
    r0 = i * br
    n1b = n1_ref[pl.ds(r0, br), :]
    n2b = n2_ref[pl.ds(r0, br), :]
    dn = (((1,), (1,)), ((), ()))
    m1 = lax.dot_general(n1b, n2_ref[...], dn, preferred_element_type=jnp.float32)
    m2 = lax.dot_general(n2b, n1_ref[...], dn, preferred_element_type=jnp.float32)
    a_val = jnp.maximum(jnp.tanh(jnp.float32(ALPHA) * (m1 - m2)), jnp.float32(0.0))
    v = a_val + _noise_tile(r0, br, n)
    vb = lax.bitcast_convert_type(v, jnp.int32)

    def rowcount(mask_bool):
        return jnp.sum(mask_bool.astype(jnp.int32), axis=1, keepdims=True)

    one_bits = jnp.int32(_EXP_ONE)
    c1 = rowcount(vb >= one_bits)
    sat = c1 >= TOPK
    lo0 = jnp.where(sat, one_bits, 0)
    hi0 = jnp.where(sat, jnp.int32(_HI_INIT), one_bits - 1)

    cl0 = jnp.where(sat, c1, jnp.int32(n))
    ch0 = jnp.zeros((br, 1), jnp.int32)
    ch0 = jnp.where(sat, ch0, c1)

    def bs_step(lo, hi, cl, ch):
        mid = (lo + hi + 1) >> 1
        cnt = rowcount(vb >= mid)
        ge = cnt >= TOPK
        lo = jnp.where(ge, mid, lo)
        cl = jnp.where(ge, cnt, cl)
        hi = jnp.where(ge, hi, mid - 1)
        ch = jnp.where(ge, ch, cnt)
        return lo, hi, cl, ch

    def bs_body17(_, carry):
        return bs_step(*carry)

    carry1 = lax.fori_loop(0, 17, bs_body17, (lo0, hi0, cl0, ch0), unroll=True)

    def bs_cond(carry):
        lo, hi, _, _ = carry
        return jnp.any(lo < hi)

    def bs_body(carry):
        return bs_step(*carry)

    tbits, _, cl, ch = lax.while_loop(bs_cond, bs_body, carry1)

    c_gt = ch
    need = TOPK - c_gt
    eq = vb == tbits
    e_cnt = cl - ch
    col = lax.broadcasted_iota(jnp.int32, (br, n), 1)
    need_eff = jnp.where(e_cnt == need, 0, need)

    def tie_cond(carry):
        _, cnt = carry
        return jnp.any(cnt < need_eff)

    def tie_body(carry):
        last, cnt = carry
        cand = jnp.where(eq & (col > last), col, jnp.int32(n))
        m = jnp.min(cand, axis=1, keepdims=True)
        active = cnt < need_eff
        return jnp.where(active, m, last), cnt + active.astype(jnp.int32)

    last0 = jnp.full((br, 1), -1, jnp.int32)
    cnt0 = jnp.zeros((br, 1), jnp.int32)
    last, _ = lax.while_loop(tie_cond, tie_body, (last0, cnt0))
    cstar = jnp.where(need_eff == 0, jnp.int32(n - 1), last)

    mask = (vb > tbits) | (eq & (col <= cstar))
    out_ref[...] = jnp.where(mask, a_val, jnp.float32(0.0))


def kernel(idx, emb1, emb2, W1, b1, W2, b2):
    n, d = emb1.shape
    g1, g2 = _sc_gather_pair(emb1, emb2, idx)

    pr_block = min(n, 1024)
    n1, n2 = pl.pallas_call(
        _prologue_body,
        grid=(n // pr_block,),
        in_specs=[
            pl.BlockSpec((pr_block, d), lambda i: (i, 0)),
            pl.BlockSpec((pr_block, d), lambda i: (i, 0)),
            pl.BlockSpec((d, d), lambda i: (0, 0)),
            pl.BlockSpec((1, d), lambda i: (0, 0)),
            pl.BlockSpec((d, d), lambda i: (0, 0)),
            pl.BlockSpec((1, d), lambda i: (0, 0)),
        ],
        out_specs=[
            pl.BlockSpec((pr_block, d), lambda i: (i, 0)),
            pl.BlockSpec((pr_block, d), lambda i: (i, 0)),
        ],
        out_shape=[
            jax.ShapeDtypeStruct((n, d), jnp.float32),
            jax.ShapeDtypeStruct((n, d), jnp.float32),
        ],
        compiler_params=pltpu.CompilerParams(
            dimension_semantics=("parallel",),
        ),
    )(g1, g2, W1, b1.reshape(1, d), W2, b2.reshape(1, d))

    br = 128
    out = pl.pallas_call(
        functools.partial(_main_body, br=br, n=n),
        grid=(n // br,),
        in_specs=[
            pl.BlockSpec((n, d), lambda i: (0, 0)),
            pl.BlockSpec((n, d), lambda i: (0, 0)),
        ],
        out_specs=pl.BlockSpec((br, n), lambda i: (i, 0)),
        out_shape=jax.ShapeDtypeStruct((n, n), jnp.float32),
        compiler_params=pltpu.CompilerParams(
            dimension_semantics=("parallel",),
            vmem_limit_bytes=100 * 1024 * 1024,
        ),
    )(n1, n2)
    return out
